# baseline (speedup 1.0000x reference)
import functools

import jax
import jax.numpy as jnp
from jax.experimental import pallas as pl

C = 16
EPS = 0.001
_STRIDES = [1, 1, 2, 1, 1, 2, 1, 1, 1, 2, 1, 1, 1, 2, 1, 1, 1]
_OUT_IDX = (4, 8, 12, 16)


def _layer_body(p0_ref, p1_ref, p2_ref, w_ref, b_ref, y_ref, *, H, W, stride):
    HW = H * W
    blocks = []
    for p_ref in (p0_ref, p1_ref, p2_ref):
        p = p_ref[0]
        z = jnp.zeros((W, C), dtype=p.dtype)
        pm = jnp.concatenate([z, p[: HW - W]], axis=0)
        pp = jnp.concatenate([p[W:], z], axis=0)
        blocks += [pm, p, pp]
    x9 = jnp.concatenate(blocks, axis=1)
    acc = jax.lax.dot_general(
        x9, w_ref[...], (((1,), (0,)), ((), ())),
        preferred_element_type=jnp.float32,
        precision=jax.lax.Precision.HIGHEST,
    )
    r = jax.lax.broadcasted_iota(jnp.int32, (HW, 1), 0)
    wcol = r % W
    zm1 = acc[:, 0:C]
    z0 = acc[:, C:2 * C]
    zp1 = acc[:, 2 * C:3 * C]
    zrow = jnp.zeros((1, C), dtype=acc.dtype)
    sm = jnp.concatenate([zrow, zm1[:-1]], axis=0)
    sp = jnp.concatenate([zp1[1:], zrow], axis=0)
    y = z0 + jnp.where(wcol == 0, 0.0, sm) + jnp.where(wcol == W - 1, 0.0, sp)
    y = jnp.maximum(y + b_ref[0], 0.0)
    y_ref[0] = y


def _conv_layer(xpad, w9, b, D_out, H, W, stride):
    HW = H * W
    body = functools.partial(_layer_body, H=H, W=W, stride=stride)
    s = stride
    return pl.pallas_call(
        body,
        grid=(D_out,),
        in_specs=[
            pl.BlockSpec((1, HW, C), lambda d: (s * d, 0, 0)),
            pl.BlockSpec((1, HW, C), lambda d: (s * d + 1, 0, 0)),
            pl.BlockSpec((1, HW, C), lambda d: (s * d + 2, 0, 0)),
            pl.BlockSpec((9 * C, 3 * C), lambda d: (0, 0)),
            pl.BlockSpec((1, C), lambda d: (0, 0)),
        ],
        out_specs=pl.BlockSpec((1, HW, C), lambda d: (d, 0, 0)),
        out_shape=jax.ShapeDtypeStruct((D_out, HW, C), jnp.float32),
    )(xpad, xpad, xpad, w9, b)


def kernel(x, params):
    inv = 1.0 / jnp.sqrt(1.0 + EPS)
    h = jnp.transpose(x[0], (1, 2, 3, 0)).reshape(64, 64 * 64, C)
    D = H = W = 64
    outs = []
    for i, ((w, g, b), s) in enumerate(zip(params, _STRIDES)):
        wS = w * (inv * g)[:, None, None, None, None]
        w9 = jnp.transpose(wS, (2, 3, 1, 4, 0)).reshape(9 * C, 3 * C)
        xpad = jnp.pad(h, ((1, 1), (0, 0), (0, 0)))
        D_out = D if s == 1 else D // 2
        y = _conv_layer(xpad, w9, b.reshape(1, C), D_out, H, W, s)
        if s == 2:
            y = y.reshape(D_out, H, W, C)[:, ::2, ::2]
            H //= 2
            W //= 2
            y = y.reshape(D_out, H * W, C)
        D = D_out
        h = y
        if i in _OUT_IDX:
            outs.append(jnp.transpose(h.reshape(D, H, W, C), (3, 0, 1, 2))[None])
    return tuple(outs)
